# Initial kernel scaffold; baseline (speedup 1.0000x reference)
import jax
import jax.numpy as jnp
from jax.experimental import pallas as pl


def kernel(tokens, lengths, emb, Wb, bb, Wbs, bbs, Wu, bu, Wus, bus, Wo, bo):
    raise NotImplementedError("write your pallas kernel here")


# speedup vs baseline: 21.4434x; 21.4434x over previous
import functools

import jax
import jax.numpy as jnp
from jax import lax
from jax.experimental import pallas as pl
from jax.experimental.pallas import tpu as pltpu
from jax.experimental.pallas import tpu_sc as plsc

_D = 128


def _sc_gather(emb, tokens):
    (B,) = tokens.shape
    V, D = emb.shape
    info = plsc.get_sparse_core_info()
    nw = info.num_cores * info.num_subcores
    bpw = B // nw
    mesh = plsc.VectorSubcoreMesh(core_axis_name="c", subcore_axis_name="s")

    @functools.partial(
        pl.kernel,
        mesh=mesh,
        out_type=jax.ShapeDtypeStruct((B, D), jnp.float32),
        scratch_types=[
            pltpu.VMEM((bpw,), jnp.int32),
            pltpu.VMEM((bpw, D), jnp.float32),
            pltpu.SemaphoreType.DMA,
        ],
    )
    def gather_k(idx_hbm, table_hbm, out_hbm, idx_v, rows_v, sem):
        wid = lax.axis_index("s") * info.num_cores + lax.axis_index("c")
        base = wid * bpw
        pltpu.sync_copy(idx_hbm.at[pl.ds(base, bpw)], idx_v)
        pltpu.async_copy(table_hbm.at[idx_v], rows_v, sem).wait()
        pltpu.sync_copy(rows_v, out_hbm.at[pl.ds(base, bpw)])

    return gather_k(tokens, emb)


def _level_body(s_in, x_ref, *refs):
    if s_in > 0:
        mm_ref = refs[0]
        refs = refs[1:]
    (wb_ref, bb_ref, wbs_ref, bbs_ref, wu_ref, bu_ref, wus_ref, bus_ref,
     h_out_ref, m_out_ref) = refs
    x = x_ref[...]
    g = jnp.dot(x, wb_ref[...], preferred_element_type=jnp.float32) + bb_ref[...]
    i = jax.nn.sigmoid(g[:, 0:_D])
    f1 = jax.nn.sigmoid(g[:, _D:2 * _D])
    f2 = jax.nn.sigmoid(g[:, 2 * _D:3 * _D])
    o = jax.nn.sigmoid(g[:, 3 * _D:4 * _D])
    u = jnp.tanh(g[:, 4 * _D:5 * _D])
    c = i * u
    if s_in > 0:
        mmv = mm_ref[...]
        m1 = mmv[:, : _D * s_in]
        m2 = mmv[:, _D * s_in:]
        c = c + f1 * m1[:, :_D] + f2 * m2[:, :_D]
    hb = o * jnp.tanh(c)
    alpha = jax.nn.sigmoid(
        jnp.dot(x, wbs_ref[...], preferred_element_type=jnp.float32) + bbs_ref[...])
    if s_in > 0:
        k = min(s_in, 3)
        al = jnp.concatenate([alpha] * k, axis=1) if k > 1 else alpha
        merged = al * m1[:, : _D * k] + (1.0 - al) * m2[:, : _D * k]
        mb = jnp.concatenate([c, merged], axis=1)
    else:
        mb = c
    gu = jnp.dot(hb, wu_ref[...], preferred_element_type=jnp.float32) + bu_ref[...]
    iu = jax.nn.sigmoid(gu[:, 0:_D])
    fu = jax.nn.sigmoid(gu[:, _D:2 * _D])
    ou = jax.nn.sigmoid(gu[:, 2 * _D:3 * _D])
    uu = jnp.tanh(gu[:, 3 * _D:4 * _D])
    cu = iu * uu + fu * mb[:, :_D]
    hu = ou * jnp.tanh(cu)
    beta = jax.nn.sigmoid(
        jnp.dot(hb, wus_ref[...], preferred_element_type=jnp.float32) + bus_ref[...])
    kp = min(mb.shape[1] // _D, 3)
    be = jnp.concatenate([beta] * kp, axis=1) if kp > 1 else beta
    pushed = be * mb[:, : _D * kp]
    h_out_ref[...] = hu
    m_out_ref[...] = jnp.concatenate([cu, pushed], axis=1)


def _level_call(x2, mm, s_in, Wb, bb2, Wbs, bbs2, Wu, bu2, Wus, bus2):
    n = x2.shape[0]
    s_out = min(s_in + 2, 4)
    blk = min(n, 1024)
    grid = (n // blk,)

    def row_spec(w):
        return pl.BlockSpec((blk, w), lambda i: (i, 0))

    def full_spec(a):
        return pl.BlockSpec(a.shape, lambda i: (0, 0))

    in_specs = [row_spec(2 * _D)]
    args = [x2]
    if s_in > 0:
        in_specs.append(row_spec(2 * _D * s_in))
        args.append(mm)
    for w in (Wb, bb2, Wbs, bbs2, Wu, bu2, Wus, bus2):
        in_specs.append(full_spec(w))
        args.append(w)

    out_shapes = (
        jax.ShapeDtypeStruct((n, _D), jnp.float32),
        jax.ShapeDtypeStruct((n, _D * s_out), jnp.float32),
    )
    out_specs = (row_spec(_D), row_spec(_D * s_out))

    return pl.pallas_call(
        functools.partial(_level_body, s_in),
        grid=grid,
        in_specs=in_specs,
        out_specs=out_specs,
        out_shape=out_shapes,
    )(*args)


def _out_body(h_ref, wo_ref, bo_ref, out_ref):
    out_ref[...] = (
        jnp.dot(h_ref[...], wo_ref[...], preferred_element_type=jnp.float32)
        + bo_ref[...])


def _out_call(h, Wo_pad, bo_pad):
    return pl.pallas_call(
        _out_body,
        out_shape=jax.ShapeDtypeStruct((h.shape[0], _D), jnp.float32),
    )(h, Wo_pad, bo_pad)


def kernel(tokens, lengths, emb, Wb, bb, Wbs, bbs, Wu, bu, Wus, bus, Wo, bo):
    del lengths
    h = _sc_gather(emb, tokens.astype(jnp.int32))
    bb2 = bb.reshape(1, -1)
    bbs2 = bbs.reshape(1, -1)
    bu2 = bu.reshape(1, -1)
    bus2 = bus.reshape(1, -1)
    m = None
    s_in = 0
    rows = h.shape[0]
    while rows > 16:
        n = rows // 2
        x2 = h.reshape(n, 2 * _D)
        mm = m.reshape(n, 2 * _D * s_in) if m is not None else None
        h, m = _level_call(x2, mm, s_in, Wb, bb2, Wbs, bbs2, Wu, bu2, Wus, bus2)
        s_in = min(s_in + 2, 4)
        rows = n
    Wo_pad = jnp.zeros((_D, _D), jnp.float32).at[:, :Wo.shape[1]].set(Wo)
    bo_pad = jnp.zeros((1, _D), jnp.float32).at[:, :bo.shape[0]].set(bo)
    out = _out_call(h, Wo_pad, bo_pad)
    logits = out[:, :Wo.shape[1]]
    return jnp.flip(logits, axis=0)


# speedup vs baseline: 50.7596x; 2.3671x over previous
import functools

import jax
import jax.numpy as jnp
import numpy as np
from jax import lax
from jax.experimental import pallas as pl
from jax.experimental.pallas import tpu as pltpu
from jax.experimental.pallas import tpu_sc as plsc

_D = 128
_B = 16
_L = 512
_LV = 9


def _bitrev_perm():
    l = np.arange(_L)
    rev = np.zeros(_L, dtype=np.int64)
    for b in range(_LV):
        rev |= ((l >> b) & 1) << (_LV - 1 - b)
    perm = np.zeros(_B * _L, dtype=np.int32)
    t = np.arange(_B)
    perm[rev[:, None] * _B + t[None, :]] = (t[None, :] * _L + l[:, None])
    return perm


def _sc_gather(emb, tokens):
    (B,) = tokens.shape
    V, D = emb.shape
    info = plsc.get_sparse_core_info()
    nw = info.num_cores * info.num_subcores
    bpw = B // nw
    mesh = plsc.VectorSubcoreMesh(core_axis_name="c", subcore_axis_name="s")

    @functools.partial(
        pl.kernel,
        mesh=mesh,
        out_type=jax.ShapeDtypeStruct((B, D), jnp.float32),
        scratch_types=[
            pltpu.VMEM((bpw,), jnp.int32),
            pltpu.VMEM((bpw, D), jnp.float32),
            pltpu.SemaphoreType.DMA,
        ],
    )
    def gather_k(idx_hbm, table_hbm, out_hbm, idx_v, rows_v, sem):
        wid = lax.axis_index("s") * info.num_cores + lax.axis_index("c")
        base = wid * bpw
        pltpu.sync_copy(idx_hbm.at[pl.ds(base, bpw)], idx_v)
        pltpu.async_copy(table_hbm.at[idx_v], rows_v, sem).wait()
        pltpu.sync_copy(rows_v, out_hbm.at[pl.ds(base, bpw)])

    return gather_k(tokens, emb)


def _tree_body(h_ref, wb_ref, bb_ref, wbs_ref, bbs_ref, wu_ref, bu_ref,
               wus_ref, bus_ref, wo_ref, bo_ref, out_ref):
    h = h_ref[...]
    m = None
    s_in = 0
    rows = h.shape[0]
    while rows > _B:
        half = rows // 2
        x = jnp.concatenate([h[:half], h[half:]], axis=1)
        g = jnp.dot(x, wb_ref[...], preferred_element_type=jnp.float32) + bb_ref[...]
        i = jax.nn.sigmoid(g[:, 0:_D])
        f1 = jax.nn.sigmoid(g[:, _D:2 * _D])
        f2 = jax.nn.sigmoid(g[:, 2 * _D:3 * _D])
        o = jax.nn.sigmoid(g[:, 3 * _D:4 * _D])
        u = jnp.tanh(g[:, 4 * _D:5 * _D])
        c = i * u
        if s_in > 0:
            m1 = m[:half]
            m2 = m[half:]
            c = c + f1 * m1[:, :_D] + f2 * m2[:, :_D]
        hb = o * jnp.tanh(c)
        alpha = jax.nn.sigmoid(
            jnp.dot(x, wbs_ref[...], preferred_element_type=jnp.float32)
            + bbs_ref[...])
        if s_in > 0:
            k = min(s_in, 3)
            al = jnp.concatenate([alpha] * k, axis=1) if k > 1 else alpha
            merged = al * m1[:, : _D * k] + (1.0 - al) * m2[:, : _D * k]
            mb = jnp.concatenate([c, merged], axis=1)
        else:
            mb = c
        gu = jnp.dot(hb, wu_ref[...], preferred_element_type=jnp.float32) + bu_ref[...]
        iu = jax.nn.sigmoid(gu[:, 0:_D])
        fu = jax.nn.sigmoid(gu[:, _D:2 * _D])
        ou = jax.nn.sigmoid(gu[:, 2 * _D:3 * _D])
        uu = jnp.tanh(gu[:, 3 * _D:4 * _D])
        cu = iu * uu + fu * mb[:, :_D]
        hu = ou * jnp.tanh(cu)
        beta = jax.nn.sigmoid(
            jnp.dot(hb, wus_ref[...], preferred_element_type=jnp.float32)
            + bus_ref[...])
        kp = min(mb.shape[1] // _D, 3)
        be = jnp.concatenate([beta] * kp, axis=1) if kp > 1 else beta
        h = hu
        m = jnp.concatenate([cu, be * mb[:, : _D * kp]], axis=1)
        s_in = min(s_in + 2, 4)
        rows = half
    out_ref[...] = (
        jnp.dot(h, wo_ref[...], preferred_element_type=jnp.float32) + bo_ref[...])


def kernel(tokens, lengths, emb, Wb, bb, Wbs, bbs, Wu, bu, Wus, bus, Wo, bo):
    del lengths
    perm = jnp.asarray(_bitrev_perm())
    tokens_perm = jnp.take(tokens.astype(jnp.int32), perm)
    h0 = _sc_gather(emb, tokens_perm)
    Wo_pad = jnp.zeros((_D, _D), jnp.float32).at[:, :Wo.shape[1]].set(Wo)
    bo_pad = jnp.zeros((1, _D), jnp.float32).at[:, :bo.shape[0]].set(bo)
    out = pl.pallas_call(
        _tree_body,
        out_shape=jax.ShapeDtypeStruct((_B, _D), jnp.float32),
    )(h0, Wb, bb.reshape(1, -1), Wbs, bbs.reshape(1, -1),
      Wu, bu.reshape(1, -1), Wus, bus.reshape(1, -1), Wo_pad, bo_pad)
    logits = out[:, :Wo.shape[1]]
    return jnp.flip(logits, axis=0)


# speedup vs baseline: 53.9738x; 1.0633x over previous
import functools

import jax
import jax.numpy as jnp
import numpy as np
from jax import lax
from jax.experimental import pallas as pl
from jax.experimental.pallas import tpu as pltpu
from jax.experimental.pallas import tpu_sc as plsc

_D = 128
_B = 16
_L = 512
_LV = 9


def _bitrev_perm():
    l = np.arange(_L)
    rev = np.zeros(_L, dtype=np.int64)
    for b in range(_LV):
        rev |= ((l >> b) & 1) << (_LV - 1 - b)
    perm = np.zeros(_B * _L, dtype=np.int32)
    t = np.arange(_B)
    perm[rev[:, None] * _B + (_B - 1 - t)[None, :]] = (t[None, :] * _L + l[:, None])
    return perm


def _sc_gather(emb, tokens, perm):
    (B,) = tokens.shape
    V, D = emb.shape
    info = plsc.get_sparse_core_info()
    nw = info.num_cores * info.num_subcores
    bpw = B // nw
    mesh = plsc.VectorSubcoreMesh(core_axis_name="c", subcore_axis_name="s")

    @functools.partial(
        pl.kernel,
        mesh=mesh,
        out_type=jax.ShapeDtypeStruct((B, D), jnp.float32),
        scratch_types=[
            pltpu.VMEM((bpw,), jnp.int32),
            pltpu.VMEM((bpw,), jnp.int32),
            pltpu.VMEM((bpw, D), jnp.float32),
            pltpu.SemaphoreType.DMA,
        ],
    )
    def gather_k(perm_hbm, tok_hbm, table_hbm, out_hbm, perm_v, idx_v, rows_v,
                 sem):
        wid = lax.axis_index("s") * info.num_cores + lax.axis_index("c")
        base = wid * bpw
        pltpu.sync_copy(perm_hbm.at[pl.ds(base, bpw)], perm_v)
        pltpu.async_copy(tok_hbm.at[perm_v], idx_v, sem).wait()
        pltpu.async_copy(table_hbm.at[idx_v], rows_v, sem).wait()
        pltpu.sync_copy(rows_v, out_hbm.at[pl.ds(base, bpw)])

    return gather_k(perm, tokens, emb)


def _sig(v):
    return 0.5 * jnp.tanh(0.5 * v) + 0.5


def _tree_body(h_ref, wb_ref, bb_ref, wbs_ref, bbs_ref, wu_ref, bu_ref,
               wus_ref, bus_ref, wo_ref, bo_ref, out_ref):
    f32 = jnp.float32
    h = h_ref[...]
    m = []
    rows = h.shape[0]
    while rows > _B:
        half = rows // 2
        h1 = h[:half]
        h2 = h[half:]
        g = (jnp.dot(h1, wb_ref[0:_D, :], preferred_element_type=f32)
             + jnp.dot(h2, wb_ref[_D:2 * _D, :], preferred_element_type=f32)
             + bb_ref[...])
        i = _sig(g[:, 0:_D])
        f1 = _sig(g[:, _D:2 * _D])
        f2 = _sig(g[:, 2 * _D:3 * _D])
        o = _sig(g[:, 3 * _D:4 * _D])
        u = jnp.tanh(g[:, 4 * _D:5 * _D])
        c = i * u
        if m:
            m1 = [s[:half] for s in m]
            m2 = [s[half:] for s in m]
            c = c + f1 * m1[0] + f2 * m2[0]
        hb = o * jnp.tanh(c)
        alpha = _sig(
            jnp.dot(h1, wbs_ref[0:_D, :], preferred_element_type=f32)
            + jnp.dot(h2, wbs_ref[_D:2 * _D, :], preferred_element_type=f32)
            + bbs_ref[...])
        mb = [c]
        if m:
            mb += [alpha * a + (1.0 - alpha) * b
                   for a, b in zip(m1[:3], m2[:3])]
        gu = (jnp.dot(hb, wu_ref[...], preferred_element_type=f32)
              + bu_ref[...])
        iu = _sig(gu[:, 0:_D])
        fu = _sig(gu[:, _D:2 * _D])
        ou = _sig(gu[:, 2 * _D:3 * _D])
        uu = jnp.tanh(gu[:, 3 * _D:4 * _D])
        cu = iu * uu + fu * mb[0]
        hu = ou * jnp.tanh(cu)
        beta = _sig(
            jnp.dot(hb, wus_ref[...], preferred_element_type=f32)
            + bus_ref[...])
        m = [cu] + [beta * s for s in mb[:3]]
        h = hu
        rows = half
    out_ref[...] = (
        jnp.dot(h, wo_ref[...], preferred_element_type=f32) + bo_ref[...])


def kernel(tokens, lengths, emb, Wb, bb, Wbs, bbs, Wu, bu, Wus, bus, Wo, bo):
    del lengths
    perm = jnp.asarray(_bitrev_perm())
    h0 = _sc_gather(emb, tokens.astype(jnp.int32), perm)
    logits = pl.pallas_call(
        _tree_body,
        out_shape=jax.ShapeDtypeStruct((_B, Wo.shape[1]), jnp.float32),
    )(h0, Wb, bb.reshape(1, -1), Wbs, bbs.reshape(1, -1),
      Wu, bu.reshape(1, -1), Wus, bus.reshape(1, -1), Wo, bo.reshape(1, -1))
    return logits
